# baseline (speedup 1.0000x reference)
import jax
import jax.numpy as jnp
from jax import lax
from jax.experimental import pallas as pl

_L = 512
_K = 64
_DM = 256
_DP = 128
_DS = 32
_NC = 32
_BR = 8
_BLK = 64

_F32 = jnp.float32
_HI = lax.Precision.HIGHEST
_SIG = (22.0 - 2.0) / 36.0


def _ln(x, g, b, eps=1e-5):
    mu = jnp.mean(x, axis=-1, keepdims=True)
    var = jnp.mean((x - mu) * (x - mu), axis=-1, keepdims=True)
    return (x - mu) / jnp.sqrt(var + eps) * g + b


def _dot(a, b):
    return jnp.dot(a, b, preferred_element_type=_F32, precision=_HI)


def _prep_body(msa, seq, state, xyz9, cat,
               gm, bm, gs, bs, wm, wsq, wst, be, gn, bn,
               node_o, nbrt_o, v0_o):
    msa_n = _ln(msa[...], gm[...], bm[...])
    st_n = _ln(state[...], gs[...], bs[...])
    pre = _dot(msa_n, wm[...]) + _dot(seq[...], wsq[...]) + _dot(st_n, wst[...]) + be[...]
    node_o[...] = _ln(pre, gn[...], bn[...])

    x9 = xyz9[...]
    ca = x9[:, 3:6]
    ca3 = jnp.concatenate([ca, ca, ca], axis=1)
    v0_o[...] = x9 - ca3

    ct = cat[...]
    dx = ca[:, 0:1] - ct[0:1, :]
    dy = ca[:, 1:2] - ct[1:2, :]
    dz = ca[:, 2:3] - ct[2:3, :]
    d = jnp.sqrt(dx * dx + dy * dy + dz * dz + 1e-8)
    rows = lax.broadcasted_iota(jnp.int32, (_L, _L), 0)
    cols = lax.broadcasted_iota(jnp.int32, (_L, _L), 1)
    d = jnp.where(rows == cols, d + 1e6, d)

    k_iota = lax.broadcasted_iota(jnp.int32, (_K, _L), 0)

    def body(k, carry):
        vals, nbrt = carry
        m = jnp.min(vals, axis=0, keepdims=True)
        am = jnp.min(jnp.where(vals == m, rows, _L), axis=0, keepdims=True)
        nbrt = jnp.where(k_iota == k, jnp.broadcast_to(am, (_K, _L)), nbrt)
        vals = jnp.where(rows == am, 1e9, vals)
        return vals, nbrt

    _, nbrt = lax.fori_loop(0, _K, body, (d, jnp.zeros((_K, _L), jnp.int32)))
    nbrt_o[...] = nbrt


def _edge_body(pair, nbrt, idxc, idxb, cax, cay, caz, caxb, cayb, cazb,
               gp, bp, w1, b1, g1, bb1, w2p, w2r, w2s, b2, g2, bb2, mu,
               e_o, unit_o):
    E = _BR * _K
    nb = nbrt[0]
    e_iota = lax.broadcasted_iota(jnp.int32, (E, 1), 0)
    selfrow = e_iota // _K
    ohs = (selfrow == lax.broadcasted_iota(jnp.int32, (E, _BR), 1)).astype(_F32)
    rowsel = ((e_iota % _K) == lax.broadcasted_iota(jnp.int32, (E, _K), 1)).astype(_F32)
    z = _dot(rowsel, nb.astype(_F32))
    nbrval = jnp.sum(z * ohs, axis=1, keepdims=True)
    nbr_i = nbrval.astype(jnp.int32)
    oh_flat = (lax.broadcasted_iota(jnp.int32, (E, _L), 1) == nbr_i).astype(_F32)

    tgt = selfrow * _L + nbr_i
    oh_big = (lax.broadcasted_iota(jnp.int32, (E, _BR * _L), 1) == tgt).astype(_F32)
    pg = _dot(oh_big, pair[...].reshape(_BR * _L, _DP))

    pn = _ln(pg, gp[...], bp[...])
    pe = _ln(_dot(pn, w1[...]) + b1[...], g1[...], bb1[...])

    idx_nbr = _dot(oh_flat, idxc[...])
    idx_self = _dot(ohs, idxb[...])
    ss = idx_nbr - idx_self
    ssf = jnp.sign(ss) * jnp.log(jnp.abs(ss) + 1.0)

    cnx = _dot(oh_flat, cax[...]); csx = _dot(ohs, caxb[...])
    cny = _dot(oh_flat, cay[...]); csy = _dot(ohs, cayb[...])
    cnz = _dot(oh_flat, caz[...]); csz = _dot(ohs, cazb[...])
    rx = cnx - csx; ry = cny - csy; rz = cnz - csz
    s2 = rx * rx + ry * ry + rz * rz
    dist = jnp.sqrt(s2 + 1e-8)
    nrm = jnp.sqrt(s2) + 1e-6
    ux = rx / nrm; uy = ry / nrm; uz = rz / nrm

    rb = jnp.exp(-(((dist - mu[...]) / _SIG) ** 2))
    e2 = _dot(pe, w2p[...]) + _dot(rb, w2r[...]) + ssf * w2s[...] + b2[...]
    e_o[...] = _ln(e2, g2[...], bb2[...])
    unit_o[...] = jnp.concatenate([ux, uy, uz], axis=1)


def _mp_body(hf, hb, vb, eb, ub, nbrt,
             wea, web, wec, be_, wh1, wh2, bh, wv,
             h_o, v_o):
    E = _BLK * _K
    e_iota = lax.broadcasted_iota(jnp.int32, (E, 1), 0)
    selfrow = e_iota // _K
    ohs = (selfrow == lax.broadcasted_iota(jnp.int32, (E, _BLK), 1)).astype(_F32)
    rowsel = ((e_iota % _K) == lax.broadcasted_iota(jnp.int32, (E, _K), 1)).astype(_F32)
    z = _dot(rowsel, nbrt[0].astype(_F32))
    nbrval = jnp.sum(z * ohs, axis=1, keepdims=True)
    nbr_i = nbrval.astype(jnp.int32)
    oh_flat = (lax.broadcasted_iota(jnp.int32, (E, _L), 1) == nbr_i).astype(_F32)

    hn = _dot(oh_flat, hf[...])
    hi_part = _dot(ohs, _dot(hb[...], wea[...]))
    m = jax.nn.relu(hi_part + _dot(hn, web[...]) + _dot(eb[...], wec[...]) + be_[...])

    seg = (lax.broadcasted_iota(jnp.int32, (_BLK, E), 0) ==
           (lax.broadcasted_iota(jnp.int32, (_BLK, E), 1) // _K)).astype(_F32)
    agg = _dot(seg, m)
    h_o[...] = hb[...] + _dot(hb[...], wh1[...]) + _dot(agg, wh2[...]) + bh[...]

    coefs = _dot(m, wv[...])
    u = ub[...]
    prods = jnp.concatenate(
        [coefs[:, a:a + 1] * u[:, c:c + 1] for a in range(3) for c in range(3)],
        axis=1)
    v_o[...] = vb[...] + _dot(seg, prods) / float(_K)


def _head_body2(h, v, ca, w0, b0, gs, bs, wl, bl, xyz_o, lddt_o):
    so = _dot(h[...], w0[...]) + b0[...]
    sn = _ln(so, gs[...], bs[...])
    lddt_o[...] = jax.nn.sigmoid(_dot(sn, wl[...]) + bl[...])
    vv = v[...]
    ca_new = ca[...] + vv[:, 3:6]
    ca_set = jnp.concatenate([ca_new, ca_new, ca_new], axis=1)
    row = lax.broadcasted_iota(jnp.int32, (_L, 1), 0)
    ca_set = jnp.where(row == 0, 0.0, ca_set)
    xyz_o[...] = vv + ca_set


def _full(shape):
    return pl.BlockSpec(shape, lambda i: tuple(0 for _ in shape))


def kernel(xyz, state, atom_coor, atom_feat, msa, pair, seq1hot, params, idx, CA_atom_index, top_k):
    B = msa.shape[0]
    xyz9 = xyz.reshape(_L, 9)
    ca = xyz[0, :, 1, :]
    ca_t = jnp.zeros((8, _L), _F32).at[0:3, :].set(ca.T)
    idxf = idx[0].astype(_F32).reshape(_L, 1)

    p = params
    r_ = lambda a: a.reshape(1, -1)
    wx, bx = p["embed_x"]
    we1, be1 = p["embed_e1"]
    we2, be2 = p["embed_e2"]
    mu = jnp.linspace(2.0, 22.0, 36).reshape(1, 36)

    node, nbrt, v0 = pl.pallas_call(
        _prep_body,
        out_shape=[
            jax.ShapeDtypeStruct((_L, _NC), _F32),
            jax.ShapeDtypeStruct((_K, _L), jnp.int32),
            jax.ShapeDtypeStruct((_L, 9), _F32),
        ],
    )(msa[0], seq1hot[0], state[0], xyz9, ca_t,
      r_(p["ln_msa"][0]), r_(p["ln_msa"][1]), r_(p["ln_state"][0]), r_(p["ln_state"][1]),
      wx[0:_DM], wx[_DM:_DM + 21], wx[_DM + 21:], r_(bx),
      r_(p["ln_node"][0]), r_(p["ln_node"][1]))

    grid_e = _L // _BR
    nbrt_e = nbrt.reshape(_K, grid_e, _BR).transpose(1, 0, 2)
    e_flat, unit = pl.pallas_call(
        _edge_body,
        grid=(grid_e,),
        in_specs=[
            pl.BlockSpec((_BR, _L, _DP), lambda i: (i, 0, 0)),
            pl.BlockSpec((1, _K, _BR), lambda i: (i, 0, 0)),
            _full((_L, 1)),
            pl.BlockSpec((_BR, 1), lambda i: (i, 0)),
            _full((_L, 1)), _full((_L, 1)), _full((_L, 1)),
            pl.BlockSpec((_BR, 1), lambda i: (i, 0)),
            pl.BlockSpec((_BR, 1), lambda i: (i, 0)),
            pl.BlockSpec((_BR, 1), lambda i: (i, 0)),
            _full((1, _DP)), _full((1, _DP)), _full((_DP, _NC)), _full((1, _NC)),
            _full((1, _NC)), _full((1, _NC)),
            _full((_NC, _NC)), _full((36, _NC)), _full((1, _NC)), _full((1, _NC)),
            _full((1, _NC)), _full((1, _NC)), _full((1, 36)),
        ],
        out_specs=[
            pl.BlockSpec((_BR * _K, _NC), lambda i: (i, 0)),
            pl.BlockSpec((_BR * _K, 3), lambda i: (i, 0)),
        ],
        out_shape=[
            jax.ShapeDtypeStruct((_L * _K, _NC), _F32),
            jax.ShapeDtypeStruct((_L * _K, 3), _F32),
        ],
    )(pair[0], nbrt_e, idxf, idxf,
      ca[:, 0:1], ca[:, 1:2], ca[:, 2:3],
      ca[:, 0:1], ca[:, 1:2], ca[:, 2:3],
      r_(p["ln_pair"][0]), r_(p["ln_pair"][1]), we1, r_(be1),
      r_(p["ln_edge1"][0]), r_(p["ln_edge1"][1]),
      we2[0:_NC], we2[_NC:_NC + 36], we2[_NC + 36:_NC + 37], r_(be2),
      r_(p["ln_edge2"][0]), r_(p["ln_edge2"][1]), mu)

    h, v = node, v0
    grid_m = _L // _BLK
    nbrt_m = nbrt.reshape(_K, grid_m, _BLK).transpose(1, 0, 2)
    for lyr in p["layers"]:
        we = lyr["We"]
        wh = lyr["Wh"]
        h, v = pl.pallas_call(
            _mp_body,
            grid=(grid_m,),
            in_specs=[
                _full((_L, _NC)),
                pl.BlockSpec((_BLK, _NC), lambda i: (i, 0)),
                pl.BlockSpec((_BLK, 9), lambda i: (i, 0)),
                pl.BlockSpec((_BLK * _K, _NC), lambda i: (i, 0)),
                pl.BlockSpec((_BLK * _K, 3), lambda i: (i, 0)),
                pl.BlockSpec((1, _K, _BLK), lambda i: (i, 0, 0)),
                _full((_NC, _NC)), _full((_NC, _NC)), _full((_NC, _NC)), _full((1, _NC)),
                _full((_NC, _NC)), _full((_NC, _NC)), _full((1, _NC)), _full((_NC, 3)),
            ],
            out_specs=[
                pl.BlockSpec((_BLK, _NC), lambda i: (i, 0)),
                pl.BlockSpec((_BLK, 9), lambda i: (i, 0)),
            ],
            out_shape=[
                jax.ShapeDtypeStruct((_L, _NC), _F32),
                jax.ShapeDtypeStruct((_L, 9), _F32),
            ],
        )(h, h, v, e_flat, unit, nbrt_m,
          we[0:_NC], we[_NC:2 * _NC], we[2 * _NC:], r_(lyr["be"]),
          wh[0:_NC], wh[_NC:], r_(lyr["bh"]), lyr["Wv"])

    w0, b0 = p["out0"]
    wl, bl = p["lddt"]
    xyz_out, lddt = pl.pallas_call(
        _head_body2,
        out_shape=[
            jax.ShapeDtypeStruct((_L, 9), _F32),
            jax.ShapeDtypeStruct((_L, 1), _F32),
        ],
    )(h, v, ca, w0, r_(b0), r_(p["ln_state"][0]), r_(p["ln_state"][1]), wl, bl.reshape(1, 1))

    return xyz_out.reshape(B * _L * 3, 3), lddt.reshape(B, _L, 1)


# speedup vs baseline: 1.6369x; 1.6369x over previous
import jax
import jax.numpy as jnp
from jax import lax
from jax.experimental import pallas as pl

_L = 512
_K = 64
_DM = 256
_DP = 128
_DS = 32
_NC = 32
_BR = 8
_BLK = 64

_F32 = jnp.float32
_HI = lax.Precision.HIGHEST
_SIG = (22.0 - 2.0) / 36.0


def _ln(x, g, b, eps=1e-5):
    mu = jnp.mean(x, axis=-1, keepdims=True)
    var = jnp.mean((x - mu) * (x - mu), axis=-1, keepdims=True)
    return (x - mu) / jnp.sqrt(var + eps) * g + b


def _dot(a, b):
    return jnp.dot(a, b, preferred_element_type=_F32, precision=_HI)


def _prep_body(msa, seq, state, xyz9, cat,
               gm, bm, gs, bs, wm, wsq, wst, be, gn, bn,
               node_o, nbrt_o, v0_o):
    msa_n = _ln(msa[...], gm[...], bm[...])
    st_n = _ln(state[...], gs[...], bs[...])
    pre = _dot(msa_n, wm[...]) + _dot(seq[...], wsq[...]) + _dot(st_n, wst[...]) + be[...]
    node_o[...] = _ln(pre, gn[...], bn[...])

    x9 = xyz9[...]
    ca = x9[:, 3:6]
    ca3 = jnp.concatenate([ca, ca, ca], axis=1)
    v0_o[...] = x9 - ca3

    ct = cat[...]
    dx = ca[:, 0:1] - ct[0:1, :]
    dy = ca[:, 1:2] - ct[1:2, :]
    dz = ca[:, 2:3] - ct[2:3, :]
    d = jnp.sqrt(dx * dx + dy * dy + dz * dz + 1e-8)
    rows = lax.broadcasted_iota(jnp.int32, (_L, _L), 0)
    cols = lax.broadcasted_iota(jnp.int32, (_L, _L), 1)
    d = jnp.where(rows == cols, d + 1e6, d)

    k_iota = lax.broadcasted_iota(jnp.int32, (_K, _L), 0)

    def body(k, carry):
        vals, nbrt = carry
        m = jnp.min(vals, axis=0, keepdims=True)
        am = jnp.min(jnp.where(vals == m, rows, _L), axis=0, keepdims=True)
        nbrt = jnp.where(k_iota == k, jnp.broadcast_to(am, (_K, _L)), nbrt)
        vals = jnp.where(rows == am, 1e9, vals)
        return vals, nbrt

    _, nbrt = lax.fori_loop(0, _K, body, (d, jnp.zeros((_K, _L), jnp.int32)))
    nbrt_o[...] = nbrt


def _edge_body(pair, nbrt, idxc, idxb, cax, cay, caz, caxb, cayb, cazb,
               gp, bp, w1, b1, g1, bb1, w2p, w2r, w2s, b2, g2, bb2, mu,
               e_o, unit_o):
    E = _BR * _K
    lane = lax.broadcasted_iota(jnp.int32, (_K, _L), 1)
    nb = nbrt[0]
    ohs_list = []
    pg_list = []
    for r in range(_BR):
        oh = (lane == nb[:, r:r + 1]).astype(_F32)
        pg_list.append(_dot(oh, pair[r]))
        ohs_list.append(oh)
    oh_flat = jnp.concatenate(ohs_list, axis=0)
    pg = jnp.concatenate(pg_list, axis=0)

    pn = _ln(pg, gp[...], bp[...])
    pe = _ln(_dot(pn, w1[...]) + b1[...], g1[...], bb1[...])

    e_iota = lax.broadcasted_iota(jnp.int32, (E, 1), 0)
    selfrow = e_iota // _K
    ohs = (selfrow == lax.broadcasted_iota(jnp.int32, (E, _BR), 1)).astype(_F32)

    idx_nbr = _dot(oh_flat, idxc[...])
    idx_self = _dot(ohs, idxb[...])
    ss = idx_nbr - idx_self
    ssf = jnp.sign(ss) * jnp.log(jnp.abs(ss) + 1.0)

    cnx = _dot(oh_flat, cax[...]); csx = _dot(ohs, caxb[...])
    cny = _dot(oh_flat, cay[...]); csy = _dot(ohs, cayb[...])
    cnz = _dot(oh_flat, caz[...]); csz = _dot(ohs, cazb[...])
    rx = cnx - csx; ry = cny - csy; rz = cnz - csz
    s2 = rx * rx + ry * ry + rz * rz
    dist = jnp.sqrt(s2 + 1e-8)
    nrm = jnp.sqrt(s2) + 1e-6
    ux = rx / nrm; uy = ry / nrm; uz = rz / nrm

    rb = jnp.exp(-(((dist - mu[...]) / _SIG) ** 2))
    e2 = _dot(pe, w2p[...]) + _dot(rb, w2r[...]) + ssf * w2s[...] + b2[...]
    e_o[...] = _ln(e2, g2[...], bb2[...])
    unit_o[...] = jnp.concatenate([ux, uy, uz], axis=1)


def _mp_body(hf, hb, vb, eb, ub, nbrt,
             wea, web, wec, be_, wh1, wh2, bh, wv,
             h_o, v_o):
    E = _BLK * _K
    e_iota = lax.broadcasted_iota(jnp.int32, (E, 1), 0)
    selfrow = e_iota // _K
    ohs = (selfrow == lax.broadcasted_iota(jnp.int32, (E, _BLK), 1)).astype(_F32)
    rowsel = ((e_iota % _K) == lax.broadcasted_iota(jnp.int32, (E, _K), 1)).astype(_F32)
    z = _dot(rowsel, nbrt[0].astype(_F32))
    nbrval = jnp.sum(z * ohs, axis=1, keepdims=True)
    nbr_i = nbrval.astype(jnp.int32)
    oh_flat = (lax.broadcasted_iota(jnp.int32, (E, _L), 1) == nbr_i).astype(_F32)

    hn = _dot(oh_flat, hf[...])
    hi_part = _dot(ohs, _dot(hb[...], wea[...]))
    m = jax.nn.relu(hi_part + _dot(hn, web[...]) + _dot(eb[...], wec[...]) + be_[...])

    seg = (lax.broadcasted_iota(jnp.int32, (_BLK, E), 0) ==
           (lax.broadcasted_iota(jnp.int32, (_BLK, E), 1) // _K)).astype(_F32)
    agg = _dot(seg, m)
    h_o[...] = hb[...] + _dot(hb[...], wh1[...]) + _dot(agg, wh2[...]) + bh[...]

    coefs = _dot(m, wv[...])
    u = ub[...]
    prods = jnp.concatenate(
        [coefs[:, a:a + 1] * u[:, c:c + 1] for a in range(3) for c in range(3)],
        axis=1)
    v_o[...] = vb[...] + _dot(seg, prods) / float(_K)


def _head_body2(h, v, ca, w0, b0, gs, bs, wl, bl, xyz_o, lddt_o):
    so = _dot(h[...], w0[...]) + b0[...]
    sn = _ln(so, gs[...], bs[...])
    lddt_o[...] = jax.nn.sigmoid(_dot(sn, wl[...]) + bl[...])
    vv = v[...]
    ca_new = ca[...] + vv[:, 3:6]
    ca_set = jnp.concatenate([ca_new, ca_new, ca_new], axis=1)
    row = lax.broadcasted_iota(jnp.int32, (_L, 1), 0)
    ca_set = jnp.where(row == 0, 0.0, ca_set)
    xyz_o[...] = vv + ca_set


def _full(shape):
    return pl.BlockSpec(shape, lambda i: tuple(0 for _ in shape))


def kernel(xyz, state, atom_coor, atom_feat, msa, pair, seq1hot, params, idx, CA_atom_index, top_k):
    B = msa.shape[0]
    xyz9 = xyz.reshape(_L, 9)
    ca = xyz[0, :, 1, :]
    ca_t = jnp.zeros((8, _L), _F32).at[0:3, :].set(ca.T)
    idxf = idx[0].astype(_F32).reshape(_L, 1)

    p = params
    r_ = lambda a: a.reshape(1, -1)
    wx, bx = p["embed_x"]
    we1, be1 = p["embed_e1"]
    we2, be2 = p["embed_e2"]
    mu = jnp.linspace(2.0, 22.0, 36).reshape(1, 36)

    node, nbrt, v0 = pl.pallas_call(
        _prep_body,
        out_shape=[
            jax.ShapeDtypeStruct((_L, _NC), _F32),
            jax.ShapeDtypeStruct((_K, _L), jnp.int32),
            jax.ShapeDtypeStruct((_L, 9), _F32),
        ],
    )(msa[0], seq1hot[0], state[0], xyz9, ca_t,
      r_(p["ln_msa"][0]), r_(p["ln_msa"][1]), r_(p["ln_state"][0]), r_(p["ln_state"][1]),
      wx[0:_DM], wx[_DM:_DM + 21], wx[_DM + 21:], r_(bx),
      r_(p["ln_node"][0]), r_(p["ln_node"][1]))

    grid_e = _L // _BR
    nbrt_e = nbrt.reshape(_K, grid_e, _BR).transpose(1, 0, 2)
    e_flat, unit = pl.pallas_call(
        _edge_body,
        grid=(grid_e,),
        in_specs=[
            pl.BlockSpec((_BR, _L, _DP), lambda i: (i, 0, 0)),
            pl.BlockSpec((1, _K, _BR), lambda i: (i, 0, 0)),
            _full((_L, 1)),
            pl.BlockSpec((_BR, 1), lambda i: (i, 0)),
            _full((_L, 1)), _full((_L, 1)), _full((_L, 1)),
            pl.BlockSpec((_BR, 1), lambda i: (i, 0)),
            pl.BlockSpec((_BR, 1), lambda i: (i, 0)),
            pl.BlockSpec((_BR, 1), lambda i: (i, 0)),
            _full((1, _DP)), _full((1, _DP)), _full((_DP, _NC)), _full((1, _NC)),
            _full((1, _NC)), _full((1, _NC)),
            _full((_NC, _NC)), _full((36, _NC)), _full((1, _NC)), _full((1, _NC)),
            _full((1, _NC)), _full((1, _NC)), _full((1, 36)),
        ],
        out_specs=[
            pl.BlockSpec((_BR * _K, _NC), lambda i: (i, 0)),
            pl.BlockSpec((_BR * _K, 3), lambda i: (i, 0)),
        ],
        out_shape=[
            jax.ShapeDtypeStruct((_L * _K, _NC), _F32),
            jax.ShapeDtypeStruct((_L * _K, 3), _F32),
        ],
    )(pair[0], nbrt_e, idxf, idxf,
      ca[:, 0:1], ca[:, 1:2], ca[:, 2:3],
      ca[:, 0:1], ca[:, 1:2], ca[:, 2:3],
      r_(p["ln_pair"][0]), r_(p["ln_pair"][1]), we1, r_(be1),
      r_(p["ln_edge1"][0]), r_(p["ln_edge1"][1]),
      we2[0:_NC], we2[_NC:_NC + 36], we2[_NC + 36:_NC + 37], r_(be2),
      r_(p["ln_edge2"][0]), r_(p["ln_edge2"][1]), mu)

    h, v = node, v0
    grid_m = _L // _BLK
    nbrt_m = nbrt.reshape(_K, grid_m, _BLK).transpose(1, 0, 2)
    for lyr in p["layers"]:
        we = lyr["We"]
        wh = lyr["Wh"]
        h, v = pl.pallas_call(
            _mp_body,
            grid=(grid_m,),
            in_specs=[
                _full((_L, _NC)),
                pl.BlockSpec((_BLK, _NC), lambda i: (i, 0)),
                pl.BlockSpec((_BLK, 9), lambda i: (i, 0)),
                pl.BlockSpec((_BLK * _K, _NC), lambda i: (i, 0)),
                pl.BlockSpec((_BLK * _K, 3), lambda i: (i, 0)),
                pl.BlockSpec((1, _K, _BLK), lambda i: (i, 0, 0)),
                _full((_NC, _NC)), _full((_NC, _NC)), _full((_NC, _NC)), _full((1, _NC)),
                _full((_NC, _NC)), _full((_NC, _NC)), _full((1, _NC)), _full((_NC, 3)),
            ],
            out_specs=[
                pl.BlockSpec((_BLK, _NC), lambda i: (i, 0)),
                pl.BlockSpec((_BLK, 9), lambda i: (i, 0)),
            ],
            out_shape=[
                jax.ShapeDtypeStruct((_L, _NC), _F32),
                jax.ShapeDtypeStruct((_L, 9), _F32),
            ],
        )(h, h, v, e_flat, unit, nbrt_m,
          we[0:_NC], we[_NC:2 * _NC], we[2 * _NC:], r_(lyr["be"]),
          wh[0:_NC], wh[_NC:], r_(lyr["bh"]), lyr["Wv"])

    w0, b0 = p["out0"]
    wl, bl = p["lddt"]
    xyz_out, lddt = pl.pallas_call(
        _head_body2,
        out_shape=[
            jax.ShapeDtypeStruct((_L, 9), _F32),
            jax.ShapeDtypeStruct((_L, 1), _F32),
        ],
    )(h, v, ca, w0, r_(b0), r_(p["ln_state"][0]), r_(p["ln_state"][1]), wl, bl.reshape(1, 1))

    return xyz_out.reshape(B * _L * 3, 3), lddt.reshape(B, _L, 1)
